# baseline (speedup 1.0000x reference)
import functools

import jax
import jax.numpy as jnp
from jax import lax
from jax.experimental import pallas as pl
from jax.experimental.pallas import tpu as pltpu
from jax.experimental.pallas import tpu_sc as plsc

N = 50000
D = 256
H = 256
E = 8
B = 256
CH = 128
NW = 32
NP = 53248
NB = NP // B
NCHW = NP // NW // CH

def _worker_id():
    return lax.axis_index("s") * 2 + lax.axis_index("c")


def _gather_body(x_hbm, gidx_hbm, out_hbm, idx_v, buf0, buf1, sem0, sem1):
    wid = _worker_id()
    pltpu.sync_copy(gidx_hbm.at[wid], idx_v)
    bufs = (buf0, buf1)
    sems = (sem0, sem1)
    cps = [pltpu.async_copy(x_hbm.at[idx_v.at[0]], buf0, sem0), None]
    for c in range(NCHW):
        if c + 1 < NCHW:
            cps[(c + 1) % 2] = pltpu.async_copy(
                x_hbm.at[idx_v.at[c + 1]], bufs[(c + 1) % 2], sems[(c + 1) % 2])
        cps[c % 2].wait()
        pltpu.sync_copy(bufs[c % 2], out_hbm.at[pl.ds((wid * NCHW + c) * CH, CH)])


@functools.cache
def _sc_kernels():
    mesh = plsc.VectorSubcoreMesh(core_axis_name="c", subcore_axis_name="s")
    gather_rows = pl.kernel(
        _gather_body,
        out_type=jax.ShapeDtypeStruct((NP, D), jnp.float32),
        mesh=mesh,
        scratch_types=[
            pltpu.VMEM((NCHW, CH), jnp.int32),
            pltpu.VMEM((CH, D), jnp.float32),
            pltpu.VMEM((CH, D), jnp.float32),
            pltpu.SemaphoreType.DMA,
            pltpu.SemaphoreType.DMA,
        ],
    )
    unperm = pl.kernel(
        _unperm_body,
        out_type=jax.ShapeDtypeStruct((NW, NCHW, CH), jnp.float32),
        mesh=mesh,
        scratch_types=[
            pltpu.VMEM((NCHW, CH), jnp.int32),
            pltpu.VMEM((NCHW, CH), jnp.int32),
            pltpu.VMEM((NCHW, CH), jnp.float32),
            pltpu.VMEM((NCHW, CH), jnp.float32),
            pltpu.SemaphoreType.DMA,
        ],
    )
    return gather_rows, unperm


def _unperm_body(y_hbm, inv_hbm, z_hbm, out_hbm, inv_v, z_v, vals, outb, gsem):
    wid = _worker_id()
    pltpu.sync_copy(inv_hbm.at[wid], inv_v)
    pltpu.sync_copy(z_hbm.at[wid], z_v)
    cps = [pltpu.async_copy(y_hbm.at[inv_v.at[c]], vals.at[c], gsem)
           for c in range(NCHW)]
    for cp in cps:
        cp.wait()
    for c in range(NCHW):
        for g in range(CH // 16):
            s = pl.ds(g * 16, 16)
            v = vals[c, s]
            zz = z_v[c, s]
            outb[c, s] = jnp.where(zz == 0, 0.0, v)
    pltpu.sync_copy(outb, out_hbm.at[wid])


def _mlp_body(eid_ref, x_ref, w1_ref, b1_ref, w2_ref, b2_ref, w3_ref, b3_ref,
              o_ref):
    x = x_ref[...]
    h = jnp.dot(x, w1_ref[0], preferred_element_type=jnp.float32) + b1_ref[0]
    h = h / (1.0 + jnp.exp(-h))
    g = jnp.dot(h, w2_ref[0], preferred_element_type=jnp.float32) + b2_ref[0]
    g = g / (1.0 + jnp.exp(-g))
    y = jnp.sum(g * w3_ref[0], axis=1)
    o_ref[0, 0] = y + b3_ref[0, 0]


def _mlp(eid, xg, w1, b1r, w2, b2r, w3r, b3r):
    grid_spec = pltpu.PrefetchScalarGridSpec(
        num_scalar_prefetch=1,
        grid=(NB,),
        in_specs=[
            pl.BlockSpec((B, D), lambda i, e: (i, 0)),
            pl.BlockSpec((1, D, H), lambda i, e: (e[i], 0, 0)),
            pl.BlockSpec((1, 1, H), lambda i, e: (e[i], 0, 0)),
            pl.BlockSpec((1, H, H), lambda i, e: (e[i], 0, 0)),
            pl.BlockSpec((1, 1, H), lambda i, e: (e[i], 0, 0)),
            pl.BlockSpec((1, 1, H), lambda i, e: (e[i], 0, 0)),
            pl.BlockSpec((1, 1, B), lambda i, e: (e[i], 0, 0)),
        ],
        out_specs=pl.BlockSpec((1, 1, B), lambda i, e: (i, 0, 0)),
    )
    return pl.pallas_call(
        _mlp_body,
        grid_spec=grid_spec,
        out_shape=jax.ShapeDtypeStruct((NB, 1, B), jnp.float32),
    )(eid, xg, w1, b1r, w2, b2r, w3r, b3r)


def kernel(X, Z, W1, b1, W2, b2, W3, b3):
    z = Z.astype(jnp.int32)
    onehot = z[:, None] == jnp.arange(1, E + 1, dtype=jnp.int32)[None, :]
    incl = jnp.cumsum(onehot.astype(jnp.int32), axis=0)
    counts = incl[-1]
    padded = ((counts + B - 1) // B) * B
    ends = jnp.cumsum(padded)
    starts = ends - padded
    rank = jnp.sum(jnp.where(onehot, incl - 1, 0), axis=1)
    valid = z > 0
    slot = jnp.where(valid, starts[jnp.clip(z - 1, 0, E - 1)] + rank, NP)
    gather_idx = jnp.zeros((NP,), jnp.int32).at[slot].set(
        jnp.arange(N, dtype=jnp.int32), mode="drop")
    inv_ext = jnp.zeros((NP,), jnp.int32).at[:N].set(
        jnp.where(valid, slot, 0).astype(jnp.int32))
    z_ext = jnp.zeros((NP,), jnp.int32).at[:N].set(z)
    blk = jnp.arange(NB, dtype=jnp.int32)
    eid = jnp.minimum(
        jnp.sum((blk[:, None] >= (ends // B)[None, :]).astype(jnp.int32),
                axis=1), E - 1).astype(jnp.int32)

    gather_rows, unperm = _sc_kernels()
    xg = gather_rows(X, gather_idx.reshape(NW, NCHW, CH))
    b1r = b1.reshape(E, 1, H)
    b2r = b2.reshape(E, 1, H)
    w3r = W3.reshape(E, 1, H)
    b3r = jnp.broadcast_to(b3.reshape(E, 1, 1), (E, 1, B))
    y = _mlp(eid, xg, W1, b1r, W2, b2r, w3r, b3r).reshape(NP)
    chi = unperm(y, inv_ext.reshape(NW, NCHW, CH),
                 z_ext.reshape(NW, NCHW, CH))
    return chi.reshape(NP)[:N]


# speedup vs baseline: 1.5635x; 1.5635x over previous
import functools

import jax
import jax.numpy as jnp
from jax import lax
from jax.experimental import pallas as pl
from jax.experimental.pallas import tpu as pltpu
from jax.experimental.pallas import tpu_sc as plsc

N = 50000
D = 256
H = 256
E = 8
B = 256
CH = 128
NW = 32
NP = 53248
NB = NP // B
NCHW = NP // NW // CH

def _worker_id():
    return lax.axis_index("s") * 2 + lax.axis_index("c")


NFULL = 390
TAIL_START = N - CH


def _route_chunk(x_hbm, slot_hbm, out_hbm, idx_v, buf, sem, start):
    pltpu.sync_copy(slot_hbm.at[pl.ds(start, CH)], idx_v)
    pltpu.sync_copy(x_hbm.at[pl.ds(start, CH)], buf)
    return pltpu.async_copy(buf, out_hbm.at[idx_v], sem)


def _route_body(x_hbm, slot_hbm, out_hbm, idx0, idx1, buf0, buf1, sem0, sem1):
    wid = _worker_id()
    idxs = (idx0, idx1)
    bufs = (buf0, buf1)
    sems = (sem0, sem1)

    @pl.when(wid <= 29)
    def _full():
        cps = [None, None]
        cps[0] = _route_chunk(x_hbm, slot_hbm, out_hbm, idx0, buf0, sem0,
                              wid * (NCHW * CH))
        for t in range(1, NCHW):
            p = t % 2
            cps[p] = _route_chunk(x_hbm, slot_hbm, out_hbm, idxs[p], bufs[p],
                                  sems[p], wid * (NCHW * CH) + t * CH)
            cps[(t + 1) % 2].wait()
        cps[(NCHW + 1) % 2].wait()

    @pl.when(wid == 30)
    def _tail():
        _route_chunk(x_hbm, slot_hbm, out_hbm, idx0, buf0, sem0,
                     TAIL_START).wait()


@functools.cache
def _sc_kernels():
    mesh = plsc.VectorSubcoreMesh(core_axis_name="c", subcore_axis_name="s")
    route_rows = pl.kernel(
        _route_body,
        out_type=jax.ShapeDtypeStruct((NP, D), jnp.float32),
        mesh=mesh,
        scratch_types=[
            pltpu.VMEM((CH,), jnp.int32),
            pltpu.VMEM((CH,), jnp.int32),
            pltpu.VMEM((CH, D), jnp.float32),
            pltpu.VMEM((CH, D), jnp.float32),
            pltpu.SemaphoreType.DMA,
            pltpu.SemaphoreType.DMA,
        ],
    )
    unperm = pl.kernel(
        _unperm_body,
        out_type=jax.ShapeDtypeStruct((NW, NCHW, CH), jnp.float32),
        mesh=mesh,
        scratch_types=[
            pltpu.VMEM((NCHW, CH), jnp.int32),
            pltpu.VMEM((NCHW, CH), jnp.int32),
            pltpu.VMEM((NCHW, CH), jnp.float32),
            pltpu.VMEM((NCHW, CH), jnp.float32),
            pltpu.SemaphoreType.DMA,
        ],
    )
    return route_rows, unperm


def _unperm_body(y_hbm, inv_hbm, z_hbm, out_hbm, inv_v, z_v, vals, outb, gsem):
    wid = _worker_id()
    pltpu.sync_copy(inv_hbm.at[wid], inv_v)
    pltpu.sync_copy(z_hbm.at[wid], z_v)
    cps = [pltpu.async_copy(y_hbm.at[inv_v.at[c]], vals.at[c], gsem)
           for c in range(NCHW)]
    for cp in cps:
        cp.wait()
    for c in range(NCHW):
        for g in range(CH // 16):
            s = pl.ds(g * 16, 16)
            v = vals[c, s]
            zz = z_v[c, s]
            outb[c, s] = jnp.where(zz == 0, 0.0, v)
    pltpu.sync_copy(outb, out_hbm.at[wid])


def _mlp_body(eid_ref, x_ref, w1_ref, b1_ref, w2_ref, b2_ref, w3_ref, b3_ref,
              o_ref):
    x = x_ref[...]
    h = jnp.dot(x, w1_ref[0], preferred_element_type=jnp.float32) + b1_ref[0]
    h = h / (1.0 + jnp.exp(-h))
    g = jnp.dot(h, w2_ref[0], preferred_element_type=jnp.float32) + b2_ref[0]
    g = g / (1.0 + jnp.exp(-g))
    y = jnp.sum(g * w3_ref[0], axis=1)
    o_ref[0, 0] = y + b3_ref[0, 0]


def _mlp(eid, xg, w1, b1r, w2, b2r, w3r, b3r):
    grid_spec = pltpu.PrefetchScalarGridSpec(
        num_scalar_prefetch=1,
        grid=(NB,),
        in_specs=[
            pl.BlockSpec((B, D), lambda i, e: (i, 0)),
            pl.BlockSpec((1, D, H), lambda i, e: (e[i], 0, 0)),
            pl.BlockSpec((1, 1, H), lambda i, e: (e[i], 0, 0)),
            pl.BlockSpec((1, H, H), lambda i, e: (e[i], 0, 0)),
            pl.BlockSpec((1, 1, H), lambda i, e: (e[i], 0, 0)),
            pl.BlockSpec((1, 1, H), lambda i, e: (e[i], 0, 0)),
            pl.BlockSpec((1, 1, B), lambda i, e: (e[i], 0, 0)),
        ],
        out_specs=pl.BlockSpec((1, 1, B), lambda i, e: (i, 0, 0)),
    )
    return pl.pallas_call(
        _mlp_body,
        grid_spec=grid_spec,
        out_shape=jax.ShapeDtypeStruct((NB, 1, B), jnp.float32),
    )(eid, xg, w1, b1r, w2, b2r, w3r, b3r)


def kernel(X, Z, W1, b1, W2, b2, W3, b3):
    z = Z.astype(jnp.int32)
    onehot = z[:, None] == jnp.arange(1, E + 1, dtype=jnp.int32)[None, :]
    incl = jnp.cumsum(onehot.astype(jnp.int32), axis=0)
    counts = incl[-1]
    padded = ((counts + B - 1) // B) * B
    ends = jnp.cumsum(padded)
    starts = ends - padded
    rank = jnp.sum(jnp.where(onehot, incl - 1, 0), axis=1)
    valid = z > 0
    slot = jnp.where(valid, starts[jnp.clip(z - 1, 0, E - 1)] + rank,
                     NP - 1).astype(jnp.int32)
    inv_ext = jnp.zeros((NP,), jnp.int32).at[:N].set(
        jnp.where(valid, slot, 0).astype(jnp.int32))
    z_ext = jnp.zeros((NP,), jnp.int32).at[:N].set(z)
    blk = jnp.arange(NB, dtype=jnp.int32)
    eid = jnp.minimum(
        jnp.sum((blk[:, None] >= (ends // B)[None, :]).astype(jnp.int32),
                axis=1), E - 1).astype(jnp.int32)

    route_rows, unperm = _sc_kernels()
    xg = route_rows(X, slot)
    b1r = b1.reshape(E, 1, H)
    b2r = b2.reshape(E, 1, H)
    w3r = W3.reshape(E, 1, H)
    b3r = jnp.broadcast_to(b3.reshape(E, 1, 1), (E, 1, B))
    y = _mlp(eid, xg, W1, b1r, W2, b2r, w3r, b3r).reshape(NP)
    chi = unperm(y, inv_ext.reshape(NW, NCHW, CH),
                 z_ext.reshape(NW, NCHW, CH))
    return chi.reshape(NP)[:N]


# speedup vs baseline: 1.8050x; 1.1545x over previous
import functools

import jax
import jax.numpy as jnp
from jax import lax
from jax.experimental import pallas as pl
from jax.experimental.pallas import tpu as pltpu
from jax.experimental.pallas import tpu_sc as plsc

N = 50000
D = 256
H = 256
E = 8
B = 256
CH = 128
NW = 32
NP = 53248
NB = NP // B
NCHW = NP // NW // CH

def _worker_id():
    return lax.axis_index("s") * 2 + lax.axis_index("c")


NFULL = 390
TAIL_START = N - CH
DW = D // 2
NBUF = 3


def _route_body(x_hbm, slot_hbm, out_hbm, idx0, idx1, idx2, buf0, buf1, buf2,
                rs0, rs1, rs2, ss0, ss1, ss2):
    wid = _worker_id()
    idxs = (idx0, idx1, idx2)
    bufs = (buf0, buf1, buf2)
    rsems = (rs0, rs1, rs2)
    ssems = (ss0, ss1, ss2)

    @pl.when(wid <= 29)
    def _full():
        base = wid * (NCHW * CH)
        reads = [None] * NCHW
        scats = [None] * NCHW

        def issue_reads(k):
            p = k % NBUF
            reads[k] = (
                pltpu.async_copy(slot_hbm.at[pl.ds(base + k * CH, CH)],
                                 idxs[p], rsems[p]),
                pltpu.async_copy(x_hbm.at[pl.ds(base + k * CH, CH)],
                                 bufs[p], rsems[p]),
            )

        for k in range(NBUF - 1):
            issue_reads(k)
        for t in range(NCHW):
            p = t % NBUF
            reads[t][0].wait()
            reads[t][1].wait()
            scats[t] = pltpu.async_copy(bufs[p], out_hbm.at[idxs[p]], ssems[p])
            nxt = t + NBUF - 1
            if nxt < NCHW:
                prev = nxt - NBUF
                if prev >= 0:
                    scats[prev] = scats[prev].wait()
                issue_reads(nxt)
        for t in range(NCHW):
            if scats[t] is not None:
                scats[t].wait()

    @pl.when(wid == 30)
    def _tail():
        pltpu.sync_copy(slot_hbm.at[pl.ds(TAIL_START, CH)], idx0)
        pltpu.sync_copy(x_hbm.at[pl.ds(TAIL_START, CH)], buf0)
        pltpu.async_copy(buf0, out_hbm.at[idx0], ss0).wait()


@functools.cache
def _sc_kernels():
    mesh = plsc.VectorSubcoreMesh(core_axis_name="c", subcore_axis_name="s")
    route_rows = pl.kernel(
        _route_body,
        out_type=jax.ShapeDtypeStruct((NP, D), jnp.float32),
        mesh=mesh,
        scratch_types=(
            [pltpu.VMEM((CH,), jnp.int32)] * NBUF
            + [pltpu.VMEM((CH, D), jnp.float32)] * NBUF
            + [pltpu.SemaphoreType.DMA] * (2 * NBUF)
        ),
    )
    unperm = pl.kernel(
        _unperm_body,
        out_type=jax.ShapeDtypeStruct((NW, NCHW, CH), jnp.float32),
        mesh=mesh,
        scratch_types=[
            pltpu.VMEM((NCHW, CH), jnp.int32),
            pltpu.VMEM((NCHW, CH), jnp.int32),
            pltpu.VMEM((NCHW, CH), jnp.float32),
            pltpu.VMEM((NCHW, CH), jnp.float32),
            pltpu.SemaphoreType.DMA,
        ],
    )
    return route_rows, unperm


def _unperm_body(y_hbm, inv_hbm, z_hbm, out_hbm, inv_v, z_v, vals, outb, gsem):
    wid = _worker_id()
    pltpu.sync_copy(inv_hbm.at[wid], inv_v)
    pltpu.sync_copy(z_hbm.at[wid], z_v)
    cps = [pltpu.async_copy(y_hbm.at[inv_v.at[c]], vals.at[c], gsem)
           for c in range(NCHW)]
    for cp in cps:
        cp.wait()
    for c in range(NCHW):
        for g in range(CH // 16):
            s = pl.ds(g * 16, 16)
            v = vals[c, s]
            zz = z_v[c, s]
            outb[c, s] = jnp.where(zz == 0, 0.0, v)
    pltpu.sync_copy(outb, out_hbm.at[wid])


def _mlp_body(eid_ref, x_ref, w1_ref, b1_ref, w2_ref, b2_ref, w3_ref, b3_ref,
              o_ref):
    x = x_ref[...].astype(jnp.bfloat16)
    h = jnp.dot(x, w1_ref[0], preferred_element_type=jnp.float32) + b1_ref[0]
    h = h / (1.0 + jnp.exp(-h))
    g = jnp.dot(h.astype(jnp.bfloat16), w2_ref[0],
                preferred_element_type=jnp.float32) + b2_ref[0]
    g = g / (1.0 + jnp.exp(-g))
    y = jnp.dot(g.astype(jnp.bfloat16), w3_ref[0],
                preferred_element_type=jnp.float32)
    o_ref[...] = y + b3_ref[0]


def _mlp(eid, xg, w1, b1r, w2, b2r, w3r, b3r):
    grid_spec = pltpu.PrefetchScalarGridSpec(
        num_scalar_prefetch=1,
        grid=(NB,),
        in_specs=[
            pl.BlockSpec((B, D), lambda i, e: (i, 0)),
            pl.BlockSpec((1, D, H), lambda i, e: (e[i], 0, 0)),
            pl.BlockSpec((1, 1, H), lambda i, e: (e[i], 0, 0)),
            pl.BlockSpec((1, H, H), lambda i, e: (e[i], 0, 0)),
            pl.BlockSpec((1, 1, H), lambda i, e: (e[i], 0, 0)),
            pl.BlockSpec((1, H, 1), lambda i, e: (e[i], 0, 0)),
            pl.BlockSpec((1, 1, 1), lambda i, e: (e[i], 0, 0)),
        ],
        out_specs=pl.BlockSpec((B, 1), lambda i, e: (i, 0)),
    )
    return pl.pallas_call(
        _mlp_body,
        grid_spec=grid_spec,
        out_shape=jax.ShapeDtypeStruct((NP, 1), jnp.float32),
    )(eid, xg, w1, b1r, w2, b2r, w3r, b3r)


def kernel(X, Z, W1, b1, W2, b2, W3, b3):
    z = Z.astype(jnp.int32)
    onehot = z[:, None] == jnp.arange(1, E + 1, dtype=jnp.int32)[None, :]
    oh3 = onehot.astype(jnp.int32).reshape(500, 100, E)
    intra = jnp.cumsum(oh3, axis=1)
    btot = intra[:, -1, :]
    prefix = jnp.cumsum(btot, axis=0) - btot
    incl = (intra + prefix[:, None, :]).reshape(N, E)
    counts = incl[-1]
    padded = ((counts + B - 1) // B) * B
    ends = jnp.cumsum(padded)
    starts = ends - padded
    rank = jnp.sum(jnp.where(onehot, incl - 1, 0), axis=1)
    valid = z > 0
    slot = jnp.where(valid, starts[jnp.clip(z - 1, 0, E - 1)] + rank,
                     NP - 1).astype(jnp.int32)
    inv_ext = jnp.zeros((NP,), jnp.int32).at[:N].set(
        jnp.where(valid, slot, 0).astype(jnp.int32))
    z_ext = jnp.zeros((NP,), jnp.int32).at[:N].set(z)
    blk = jnp.arange(NB, dtype=jnp.int32)
    eid = jnp.minimum(
        jnp.sum((blk[:, None] >= (ends // B)[None, :]).astype(jnp.int32),
                axis=1), E - 1).astype(jnp.int32)

    route_rows, unperm = _sc_kernels()
    xg = route_rows(X, slot)
    b1r = b1.reshape(E, 1, H)
    b2r = b2.reshape(E, 1, H)
    w3r = W3.astype(jnp.bfloat16)
    b3r = b3.reshape(E, 1, 1)
    y = _mlp(eid, xg, W1.astype(jnp.bfloat16), b1r, W2.astype(jnp.bfloat16),
             b2r, w3r, b3r).reshape(NP)
    chi = unperm(y, inv_ext.reshape(NW, NCHW, CH),
                 z_ext.reshape(NW, NCHW, CH))
    return chi.reshape(NP)[:N]
